# baseline (speedup 1.0000x reference)
import math

import jax
import jax.numpy as jnp
from jax.experimental import pallas as pl
from jax.experimental.pallas import tpu as pltpu

HASH_LEN = 9
BQ = 2048

_G_COEFFS = (
    0.636598527431488,
    0.10729582607746124,
    0.034997887909412384,
    0.06604475528001785,
)


def _p9(neghalf_dot):
    z = jnp.maximum(neghalf_dot + 0.5, 1e-30)
    s = z * jax.lax.rsqrt(z)
    g = _G_COEFFS[-1]
    for c in _G_COEFFS[-2::-1]:
        g = c + z * g
    p = 1.0 - s * g
    p2 = p * p
    p4 = p2 * p2
    p8 = p4 * p4
    return p8 * p


def _yoso_block(qt_ref, kt_ref, vt_ref, mk_ref, mq_ref, o_ref):
    qt = qt_ref[0, 0] * -0.5
    kt = kt_ref[0, 0]
    vt = vt_ref[0, 0] * mk_ref[0]
    dot_t = jax.lax.dot_general(
        kt, qt, (((0,), (0,)), ((), ())), preferred_element_type=jnp.float32)
    p9_t = _p9(dot_t)
    xt = jax.lax.dot_general(
        vt, p9_t, (((1,), (0,)), ((), ())), preferred_element_type=jnp.float32)
    xt = xt * mq_ref[0]
    n2 = jnp.sum(xt * xt, axis=0, keepdims=True)
    o_ref[0, 0] = xt * jax.lax.rsqrt(n2 + 1e-24)


def kernel(Q, K, V, mask):
    B, H, S, D = Q.shape
    Qt = jnp.transpose(Q, (0, 1, 3, 2))
    Kt = jnp.transpose(K, (0, 1, 3, 2))
    Vt = jnp.transpose(V, (0, 1, 3, 2))
    mask2 = mask.astype(Q.dtype).reshape(B, 1, S)

    grid = (B * H, S // BQ)
    out_t = pl.pallas_call(
        _yoso_block,
        grid=grid,
        in_specs=[
            pl.BlockSpec((1, 1, D, BQ), lambda g, i: (g // H, g % H, 0, i)),
            pl.BlockSpec((1, 1, D, S), lambda g, i: (g // H, g % H, 0, 0)),
            pl.BlockSpec((1, 1, D, S), lambda g, i: (g // H, g % H, 0, 0)),
            pl.BlockSpec((1, 1, S), lambda g, i: (g // H, 0, 0)),
            pl.BlockSpec((1, 1, BQ), lambda g, i: (g // H, 0, i)),
        ],
        out_specs=pl.BlockSpec((1, 1, D, BQ), lambda g, i: (g // H, g % H, 0, i)),
        out_shape=jax.ShapeDtypeStruct((B, H, D, S), Q.dtype),
        compiler_params=pltpu.CompilerParams(
            dimension_semantics=("arbitrary", "arbitrary"),
        ),
    )(Qt, Kt, Vt, mask2, mask2)
    return jnp.transpose(out_t, (0, 1, 3, 2))


# speedup vs baseline: 1.0067x; 1.0067x over previous
import math

import jax
import jax.numpy as jnp
from jax.experimental import pallas as pl
from jax.experimental.pallas import tpu as pltpu

HASH_LEN = 9
BQ = 2048

_G_COEFFS = (
    0.636598527431488,
    0.10729582607746124,
    0.034997887909412384,
    0.06604475528001785,
)


def _p9(neghalf_dot):
    z = jnp.maximum(neghalf_dot + 0.5, 1e-30)
    s = z * jax.lax.rsqrt(z)
    g = _G_COEFFS[-1]
    for c in _G_COEFFS[-2::-1]:
        g = c + z * g
    p = 1.0 - s * g
    p2 = p * p
    p4 = p2 * p2
    p8 = p4 * p4
    return p8 * p


def _yoso_block(qt_ref, kt_ref, vt_ref, mk_ref, mq_ref, o_ref):
    qt = qt_ref[0, 0] * -0.5
    kt = kt_ref[0, 0]
    vt = vt_ref[0, 0] * mk_ref[0]
    dot = jax.lax.dot_general(
        qt, kt, (((0,), (0,)), ((), ())), preferred_element_type=jnp.float32)
    p9 = _p9(dot)
    xt = jax.lax.dot_general(
        vt, p9, (((1,), (1,)), ((), ())), preferred_element_type=jnp.float32)
    xt = xt * mq_ref[0]
    n2 = jnp.sum(xt * xt, axis=0, keepdims=True)
    o_ref[0, 0] = xt * jax.lax.rsqrt(n2 + 1e-24)


def kernel(Q, K, V, mask):
    B, H, S, D = Q.shape
    Qt = jnp.transpose(Q, (0, 1, 3, 2))
    Kt = jnp.transpose(K, (0, 1, 3, 2))
    Vt = jnp.transpose(V, (0, 1, 3, 2))
    mask2 = mask.astype(Q.dtype).reshape(B, 1, S)

    grid = (B * H, S // BQ)
    out_t = pl.pallas_call(
        _yoso_block,
        grid=grid,
        in_specs=[
            pl.BlockSpec((1, 1, D, BQ), lambda g, i: (g // H, g % H, 0, i)),
            pl.BlockSpec((1, 1, D, S), lambda g, i: (g // H, g % H, 0, 0)),
            pl.BlockSpec((1, 1, D, S), lambda g, i: (g // H, g % H, 0, 0)),
            pl.BlockSpec((1, 1, S), lambda g, i: (g // H, 0, 0)),
            pl.BlockSpec((1, 1, BQ), lambda g, i: (g // H, 0, i)),
        ],
        out_specs=pl.BlockSpec((1, 1, D, BQ), lambda g, i: (g // H, g % H, 0, i)),
        out_shape=jax.ShapeDtypeStruct((B, H, D, S), Q.dtype),
        compiler_params=pltpu.CompilerParams(
            dimension_semantics=("parallel", "parallel"),
        ),
    )(Qt, Kt, Vt, mask2, mask2)
    return jnp.transpose(out_t, (0, 1, 3, 2))
